# Initial kernel scaffold; baseline (speedup 1.0000x reference)
import functools

import jax
import jax.numpy as jnp
from jax import lax
from jax.experimental import pallas as pl
from jax.experimental.pallas import tpu as pltpu
from jax.experimental.pallas import tpu_sc as plsc

N_NODES = 10000
N_EDGES = 160000
DIN = 256
DOUT = 256
HALF = DOUT // 2

NTILES = 16
E2 = 2 * N_EDGES
PER_TILE = E2 // NTILES
CHUNK = 80
NCHUNK = PER_TILE // CHUNK
DRAIN_ROWS = 125


def _matmul_body(x_ref, w_ref, o_ref):
    o_ref[...] = jnp.dot(x_ref[...], w_ref[...],
                         preferred_element_type=jnp.float32)


def _support_halves(x, w):
    bn = 1000
    return pl.pallas_call(
        _matmul_body,
        grid=(2, N_NODES // bn),
        in_specs=[
            pl.BlockSpec((bn, DIN), lambda h, i: (i, 0)),
            pl.BlockSpec((DIN, HALF), lambda h, i: (0, h)),
        ],
        out_specs=pl.BlockSpec((bn, HALF),
                               lambda h, i: (h * (N_NODES // bn) + i, 0)),
        out_shape=jax.ShapeDtypeStruct((2 * N_NODES, HALF), jnp.float32),
    )(x, w)


def _sc_body(support_hbm, src_hbm, dst_hbm, rel_hbm, alpha_hbm, bias_hbm,
             out_hbm,
             alpha_v, bias_v, srcidx_v, dstidx_v, rel_v, alp_v, rows_v,
             drain_v, accum_sh, sem_g):
    c = lax.axis_index("c")
    s = lax.axis_index("s")

    pltpu.sync_copy(alpha_hbm, alpha_v)
    pltpu.sync_copy(bias_hbm.at[pl.ds(c * HALF, HALF)], bias_v)

    zero16 = jnp.zeros((16,), jnp.float32)

    def _zrow(r, carry):
        for v in range(HALF // 16):
            drain_v[r, pl.ds(v * 16, 16)] = zero16
        return carry

    lax.fori_loop(0, DRAIN_ROWS, _zrow, 0)
    for k in range(N_NODES // NTILES // DRAIN_ROWS):
        pltpu.sync_copy(
            drain_v,
            accum_sh.at[pl.ds(s * (N_NODES // NTILES) + k * DRAIN_ROWS,
                              DRAIN_ROWS)])
    plsc.subcore_barrier()

    def _chunk(ci, carry):
        base = s * PER_TILE + ci * CHUNK
        pltpu.sync_copy(src_hbm.at[pl.ds(base, CHUNK)], srcidx_v)
        pltpu.sync_copy(dst_hbm.at[pl.ds(base, CHUNK)], dstidx_v)
        pltpu.sync_copy(rel_hbm.at[pl.ds(base, CHUNK)], rel_v)
        for g in range(CHUNK // 16):
            sl = pl.ds(g * 16, 16)
            srcidx_v[sl] = srcidx_v[sl] + c * N_NODES
            r16 = rel_v[sl]
            a = plsc.load_gather(alpha_v, [r16])
            alp_v[sl] = jnp.where(r16 == 0, jnp.zeros((16,), jnp.float32), a)
        pltpu.async_copy(support_hbm.at[srcidx_v], rows_v, sem_g).wait()

        def _scale(e, cc):
            spl = plsc.load_gather(alp_v, [jnp.broadcast_to(e, (16,))])
            for v in range(HALF // 16):
                sl = pl.ds(v * 16, 16)
                rows_v[e, sl] = rows_v[e, sl] * spl
            return cc

        lax.fori_loop(0, CHUNK, _scale, 0)
        pltpu.sync_copy(rows_v, accum_sh.at[dstidx_v], add=True)
        return carry

    lax.fori_loop(0, NCHUNK, _chunk, 0)
    plsc.subcore_barrier()

    for k in range(N_NODES // NTILES // DRAIN_ROWS):
        row0 = s * (N_NODES // NTILES) + k * DRAIN_ROWS
        pltpu.sync_copy(accum_sh.at[pl.ds(row0, DRAIN_ROWS)], drain_v)

        def _brow(r, carry):
            for v in range(HALF // 16):
                sl = pl.ds(v * 16, 16)
                drain_v[r, sl] = drain_v[r, sl] + bias_v[sl]
            return carry

        lax.fori_loop(0, DRAIN_ROWS, _brow, 0)
        pltpu.sync_copy(drain_v, out_hbm.at[pl.ds(c * N_NODES + row0,
                                                  DRAIN_ROWS)])


@functools.partial(
    pl.kernel,
    mesh=plsc.VectorSubcoreMesh(core_axis_name="c", subcore_axis_name="s"),
    out_type=jax.ShapeDtypeStruct((2 * N_NODES, HALF), jnp.float32),
    scratch_types=[
        pltpu.VMEM((32,), jnp.float32),
        pltpu.VMEM((HALF,), jnp.float32),
        pltpu.VMEM((CHUNK,), jnp.int32),
        pltpu.VMEM((CHUNK,), jnp.int32),
        pltpu.VMEM((CHUNK,), jnp.int32),
        pltpu.VMEM((CHUNK,), jnp.float32),
        pltpu.VMEM((CHUNK, HALF), jnp.float32),
        pltpu.VMEM((DRAIN_ROWS, HALF), jnp.float32),
        pltpu.VMEM_SHARED((N_NODES, HALF), jnp.float32),
        pltpu.SemaphoreType.DMA,
    ],
)
def _sc_scatter(support_hbm, src_hbm, dst_hbm, rel_hbm, alpha_hbm, bias_hbm,
                out_hbm, *scratch):
    _sc_body(support_hbm, src_hbm, dst_hbm, rel_hbm, alpha_hbm, bias_hbm,
             out_hbm, *scratch)


def kernel(input, edge_index, rel, weight, alpha_table, bias):
    support_cat = _support_halves(input, weight)
    row, col = edge_index[0], edge_index[1]
    src2 = jnp.concatenate([col, row])
    dst2 = jnp.concatenate([row, col])
    rel2 = jnp.concatenate([rel, rel])
    alpha_flat = jnp.pad(alpha_table[:, 0], (0, 32 - alpha_table.shape[0]))
    out2 = _sc_scatter(support_cat, src2, dst2, rel2, alpha_flat, bias)
    return jnp.concatenate([out2[:N_NODES], out2[N_NODES:]], axis=1)


# speedup vs baseline: 2.4291x; 2.4291x over previous
import functools

import jax
import jax.numpy as jnp
from jax import lax
from jax.experimental import pallas as pl
from jax.experimental.pallas import tpu as pltpu
from jax.experimental.pallas import tpu_sc as plsc

N_NODES = 10000
N_EDGES = 160000
DIN = 256
DOUT = 256
HALF = DOUT // 2

NTILES = 16
E2 = 2 * N_EDGES
PER_TILE = E2 // NTILES
CHUNK = 80
NCHUNK = PER_TILE // CHUNK
NPAD = 10240
ROWS_PER_TILE = NPAD // NTILES
DRAIN_ROWS = 128


def _matmul_body(x_ref, w_ref, o_ref):
    o_ref[...] = jnp.dot(x_ref[...], w_ref[...],
                         preferred_element_type=jnp.float32)


def _support_halves(x, w):
    bn = 1000
    return pl.pallas_call(
        _matmul_body,
        grid=(2, N_NODES // bn),
        in_specs=[
            pl.BlockSpec((bn, DIN), lambda h, i: (i, 0)),
            pl.BlockSpec((DIN, HALF), lambda h, i: (0, h)),
        ],
        out_specs=pl.BlockSpec((bn, HALF),
                               lambda h, i: (h * (N_NODES // bn) + i, 0)),
        out_shape=jax.ShapeDtypeStruct((2 * N_NODES, HALF), jnp.float32),
    )(x, w)


def _sc_body(support_hbm, src_hbm, dst_hbm, rel_hbm, alpha_hbm, bias_hbm,
             out_hbm,
             alpha_v, bias_v, srcidx_v, dstidx_v, rel_v, alp_v, rows_v,
             drain_v, accum_sh, sem_g):
    c = lax.axis_index("c")
    s = lax.axis_index("s")

    pltpu.sync_copy(alpha_hbm, alpha_v)
    pltpu.sync_copy(bias_hbm.at[pl.ds(c * HALF, HALF)], bias_v)

    zero16 = jnp.zeros((16,), jnp.float32)

    def _zrow(r, carry):
        for v in range(HALF // 16):
            drain_v[r, pl.ds(v * 16, 16)] = zero16
        return carry

    lax.fori_loop(0, DRAIN_ROWS, _zrow, 0)
    for k in range(ROWS_PER_TILE // DRAIN_ROWS):
        pltpu.sync_copy(
            drain_v,
            accum_sh.at[pl.ds(s * ROWS_PER_TILE + k * DRAIN_ROWS,
                              DRAIN_ROWS)])
    plsc.subcore_barrier()

    def _chunk(ci, carry):
        base = s * PER_TILE + ci * CHUNK
        pltpu.sync_copy(src_hbm.at[pl.ds(base, CHUNK)], srcidx_v)
        pltpu.sync_copy(dst_hbm.at[pl.ds(base, CHUNK)], dstidx_v)
        pltpu.sync_copy(rel_hbm.at[pl.ds(base, CHUNK)], rel_v)
        for g in range(CHUNK // 16):
            sl = pl.ds(g * 16, 16)
            srcidx_v[sl] = srcidx_v[sl] + c * N_NODES
            r16 = rel_v[sl]
            a = plsc.load_gather(alpha_v, [r16])
            alp_v[sl] = jnp.where(r16 == 0, jnp.zeros((16,), jnp.float32), a)
        pltpu.async_copy(support_hbm.at[srcidx_v], rows_v, sem_g).wait()

        def _scale(e, cc):
            spl = plsc.load_gather(alp_v, [jnp.broadcast_to(e, (16,))])
            for v in range(HALF // 16):
                sl = pl.ds(v * 16, 16)
                rows_v[e, sl] = rows_v[e, sl] * spl
            return cc

        lax.fori_loop(0, CHUNK, _scale, 0)
        pltpu.sync_copy(rows_v, accum_sh.at[dstidx_v], add=True)
        return carry

    lax.fori_loop(0, NCHUNK, _chunk, 0)
    plsc.subcore_barrier()

    for k in range(ROWS_PER_TILE // DRAIN_ROWS):
        row0 = s * ROWS_PER_TILE + k * DRAIN_ROWS
        pltpu.sync_copy(accum_sh.at[pl.ds(row0, DRAIN_ROWS)], drain_v)

        def _brow(r, carry):
            for v in range(HALF // 16):
                sl = pl.ds(v * 16, 16)
                drain_v[r, sl] = drain_v[r, sl] + bias_v[sl]
            return carry

        lax.fori_loop(0, DRAIN_ROWS, _brow, 0)
        pltpu.sync_copy(drain_v, out_hbm.at[pl.ds(c * NPAD + row0,
                                                  DRAIN_ROWS)])


@functools.partial(
    pl.kernel,
    mesh=plsc.VectorSubcoreMesh(core_axis_name="c", subcore_axis_name="s"),
    out_type=jax.ShapeDtypeStruct((2 * NPAD, HALF), jnp.float32),
    compiler_params=pltpu.CompilerParams(needs_layout_passes=False),
    scratch_types=[
        pltpu.VMEM((32,), jnp.float32),
        pltpu.VMEM((HALF,), jnp.float32),
        pltpu.VMEM((CHUNK,), jnp.int32),
        pltpu.VMEM((CHUNK,), jnp.int32),
        pltpu.VMEM((CHUNK,), jnp.int32),
        pltpu.VMEM((CHUNK,), jnp.float32),
        pltpu.VMEM((CHUNK, HALF), jnp.float32),
        pltpu.VMEM((DRAIN_ROWS, HALF), jnp.float32),
        pltpu.VMEM_SHARED((NPAD, HALF), jnp.float32),
        pltpu.SemaphoreType.DMA,
    ],
)
def _sc_scatter(support_hbm, src_hbm, dst_hbm, rel_hbm, alpha_hbm, bias_hbm,
                out_hbm, *scratch):
    _sc_body(support_hbm, src_hbm, dst_hbm, rel_hbm, alpha_hbm, bias_hbm,
             out_hbm, *scratch)


def kernel(input, edge_index, rel, weight, alpha_table, bias):
    support_cat = _support_halves(input, weight)
    row, col = edge_index[0], edge_index[1]
    src2 = jnp.concatenate([col, row])
    dst2 = jnp.concatenate([row, col])
    rel2 = jnp.concatenate([rel, rel])
    alpha_flat = jnp.pad(alpha_table[:, 0], (0, 32 - alpha_table.shape[0]))
    out2 = _sc_scatter(support_cat, src2, dst2, rel2, alpha_flat, bias)
    return jnp.concatenate([out2[:N_NODES], out2[NPAD:NPAD + N_NODES]],
                           axis=1)


# speedup vs baseline: 4.1009x; 1.6882x over previous
import functools

import jax
import jax.numpy as jnp
from jax import lax
from jax.experimental import pallas as pl
from jax.experimental.pallas import tpu as pltpu
from jax.experimental.pallas import tpu_sc as plsc

N_NODES = 10000
DIN = 256
DOUT = 256
HALF = DOUT // 2
N_EDGES = 160000

NTILES = 16
E2 = 2 * N_EDGES
PER_TILE = E2 // NTILES
CHUNK = 80
CPT = PER_TILE // CHUNK
REC = 3 * CHUNK
NPAD = 10240
ROWS_PER_TILE = NPAD // NTILES
DRAIN_CHUNKS = ROWS_PER_TILE // CHUNK


def _matmul_body(x_ref, w_ref, o_ref):
    o_ref[...] = jnp.dot(x_ref[...], w_ref[...],
                         preferred_element_type=jnp.float32)


def _support_halves(x, w):
    bn = 1000
    return pl.pallas_call(
        _matmul_body,
        grid=(2, N_NODES // bn),
        in_specs=[
            pl.BlockSpec((bn, DIN), lambda h, i: (i, 0)),
            pl.BlockSpec((DIN, HALF), lambda h, i: (0, h)),
        ],
        out_specs=pl.BlockSpec((bn, HALF),
                               lambda h, i: (h * (N_NODES // bn) + i, 0)),
        out_shape=jax.ShapeDtypeStruct((2 * N_NODES, HALF), jnp.float32),
    )(x, w)


def _sc_body(support_hbm, edata_hbm, alpha_hbm, bias_hbm, out_hbm,
             alpha_v, bias_v, ebuf, srcc, dstc, alpc, rows, accum_sh,
             si, sg, ss):
    c = lax.axis_index("c")
    s = lax.axis_index("s")
    cN = c * N_NODES

    pltpu.sync_copy(alpha_hbm, alpha_v)
    pltpu.sync_copy(bias_hbm.at[pl.ds(c * HALF, HALF)], bias_v)

    lane = lax.iota(jnp.int32, 16)
    a16 = alpha_v[pl.ds(0, 16)]
    alpha_v[pl.ds(0, 16)] = jnp.where(lane == 0,
                                      jnp.zeros((16,), jnp.float32), a16)

    zero16 = jnp.zeros((16,), jnp.float32)

    def _zrow(r, carry):
        for v in range(HALF // 16):
            rows[0][r, pl.ds(v * 16, 16)] = zero16
        return carry

    lax.fori_loop(0, CHUNK, _zrow, 0)
    for k in range(DRAIN_CHUNKS):
        pltpu.sync_copy(
            rows[0],
            accum_sh.at[pl.ds(s * ROWS_PER_TILE + k * CHUNK, CHUNK)])
    plsc.subcore_barrier()

    def idx_start(g, b):
        pltpu.async_copy(
            edata_hbm.at[pl.ds((s * CPT + g) * REC, REC)], ebuf[b], si[b])

    def idx_wait(g, b):
        pltpu.make_async_copy(
            edata_hbm.at[pl.ds((s * CPT + g) * REC, REC)],
            ebuf[b], si[b]).wait()

    def convert(b):
        for j in range(CHUNK // 16):
            sl = pl.ds(j * 16, 16)
            srcc[b][sl] = ebuf[b][pl.ds(j * 16, 16)] + cN
            dstc[b][sl] = ebuf[b][pl.ds(CHUNK + j * 16, 16)]
            r16 = ebuf[b][pl.ds(2 * CHUNK + j * 16, 16)]
            alpc[b][sl] = plsc.load_gather(alpha_v, [r16])

    def gather_start(b):
        pltpu.async_copy(support_hbm.at[srcc[b]], rows[b], sg[b])

    def gather_wait(b):
        pltpu.make_async_copy(support_hbm.at[srcc[b]], rows[b], sg[b]).wait()

    def scatter_start(b):
        pltpu.async_copy(rows[b], accum_sh.at[dstc[b]], ss[b], add=True)

    def scatter_wait(b):
        pltpu.make_async_copy(rows[b], accum_sh.at[dstc[b]], ss[b]).wait()

    def scale(b):
        def _scale(i, c2, _b=b):
            for e in range(16):
                idx = i * 16 + e
                spl = plsc.load_gather(
                    alpc[_b], [jnp.broadcast_to(idx, (16,))])
                for v in range(HALF // 16):
                    sl = pl.ds(v * 16, 16)
                    rows[_b][idx, sl] = rows[_b][idx, sl] * spl
            return c2

        lax.fori_loop(0, CHUNK // 16, _scale, 0)

    idx_start(0, 0)
    idx_wait(0, 0)
    convert(0)
    gather_start(0)
    idx_start(1, 1)

    def _outer(gi, cc):
        for b in range(2):
            g = gi * 2 + b
            ob = 1 - b
            gather_wait(b)
            scale(b)
            scatter_start(b)

            @pl.when(g + 1 < CPT)
            def _next(_b=b, _ob=ob, _g=g):
                @pl.when(_g >= 1)
                def _ws():
                    scatter_wait(_ob)

                idx_wait(_g + 1, _ob)
                convert(_ob)
                gather_start(_ob)

                @pl.when(_g + 2 < CPT)
                def _pf():
                    idx_start(_g + 2, _b)
        return cc

    lax.fori_loop(0, CPT // 2, _outer, 0)
    scatter_wait(0)
    scatter_wait(1)
    plsc.subcore_barrier()

    for k in range(DRAIN_CHUNKS):
        row0 = s * ROWS_PER_TILE + k * CHUNK
        pltpu.sync_copy(accum_sh.at[pl.ds(row0, CHUNK)], rows[0])

        def _brow(r, carry):
            for v in range(HALF // 16):
                sl = pl.ds(v * 16, 16)
                rows[0][r, sl] = rows[0][r, sl] + bias_v[sl]
            return carry

        lax.fori_loop(0, CHUNK, _brow, 0)
        pltpu.sync_copy(rows[0], out_hbm.at[pl.ds(c * NPAD + row0, CHUNK)])


@functools.partial(
    pl.kernel,
    mesh=plsc.VectorSubcoreMesh(core_axis_name="c", subcore_axis_name="s"),
    out_type=jax.ShapeDtypeStruct((2 * NPAD, HALF), jnp.float32),
    compiler_params=pltpu.CompilerParams(needs_layout_passes=False),
    scratch_types=[
        pltpu.VMEM((32,), jnp.float32),
        pltpu.VMEM((HALF,), jnp.float32),
        [pltpu.VMEM((REC,), jnp.int32)] * 2,
        [pltpu.VMEM((CHUNK,), jnp.int32)] * 2,
        [pltpu.VMEM((CHUNK,), jnp.int32)] * 2,
        [pltpu.VMEM((CHUNK,), jnp.float32)] * 2,
        [pltpu.VMEM((CHUNK, HALF), jnp.float32)] * 2,
        pltpu.VMEM_SHARED((NPAD, HALF), jnp.float32),
        [pltpu.SemaphoreType.DMA] * 2,
        [pltpu.SemaphoreType.DMA] * 2,
        [pltpu.SemaphoreType.DMA] * 2,
    ],
)
def _sc_scatter(support_hbm, edata_hbm, alpha_hbm, bias_hbm, out_hbm,
                *scratch):
    _sc_body(support_hbm, edata_hbm, alpha_hbm, bias_hbm, out_hbm, *scratch)


def kernel(input, edge_index, rel, weight, alpha_table, bias):
    support_cat = _support_halves(input, weight)
    row, col = edge_index[0], edge_index[1]
    src2 = jnp.concatenate([col, row]).reshape(-1, CHUNK)
    dst2 = jnp.concatenate([row, col]).reshape(-1, CHUNK)
    rel2 = jnp.concatenate([rel, rel]).reshape(-1, CHUNK)
    edata = jnp.stack([src2, dst2, rel2], axis=1).reshape(-1)
    alpha_flat = jnp.pad(alpha_table[:, 0], (0, 32 - alpha_table.shape[0]))
    out2 = _sc_scatter(support_cat, edata, alpha_flat, bias)
    return jnp.concatenate([out2[:N_NODES], out2[NPAD:NPAD + N_NODES]],
                           axis=1)


# speedup vs baseline: 5.3919x; 1.3148x over previous
import functools

import jax
import jax.numpy as jnp
from jax import lax
from jax.experimental import pallas as pl
from jax.experimental.pallas import tpu as pltpu
from jax.experimental.pallas import tpu_sc as plsc

N_NODES = 10000
DIN = 256
DOUT = 256
HALF = DOUT // 2
N_EDGES = 160000

NTILES = 16
E2 = 2 * N_EDGES
PER_TILE = E2 // NTILES
CHUNK = 80
CPT = PER_TILE // CHUNK
REC = 3 * CHUNK
NPAD = 10240
ROWS_PER_TILE = NPAD // NTILES
DRAIN_CHUNKS = ROWS_PER_TILE // CHUNK


def _matmul_body(x_ref, w_ref, o_ref):
    o_ref[...] = jnp.dot(x_ref[...], w_ref[...],
                         preferred_element_type=jnp.float32)


def _support_halves(x, w):
    bn = 1000
    return pl.pallas_call(
        _matmul_body,
        grid=(2, N_NODES // bn),
        in_specs=[
            pl.BlockSpec((bn, DIN), lambda h, i: (i, 0)),
            pl.BlockSpec((DIN, HALF), lambda h, i: (0, h)),
        ],
        out_specs=pl.BlockSpec((bn, HALF),
                               lambda h, i: (h * (N_NODES // bn) + i, 0)),
        out_shape=jax.ShapeDtypeStruct((2 * N_NODES, HALF), jnp.float32),
    )(x, w)


def _sc_body(support_hbm, edata_hbm, alpha_hbm, bias_hbm, out_hbm,
             alpha_v, bias_v, ebuf, srcc, dstc, alpc, rows, scaled, accum_sh,
             si, sg, ss):
    c = lax.axis_index("c")
    s = lax.axis_index("s")
    cN = c * N_NODES

    pltpu.sync_copy(alpha_hbm, alpha_v)
    pltpu.sync_copy(bias_hbm.at[pl.ds(c * HALF, HALF)], bias_v)

    lane = lax.iota(jnp.int32, 16)
    a16 = alpha_v[pl.ds(0, 16)]
    alpha_v[pl.ds(0, 16)] = jnp.where(lane == 0,
                                      jnp.zeros((16,), jnp.float32), a16)

    zero16 = jnp.zeros((16,), jnp.float32)

    def _zrow(r, carry):
        for v in range(HALF // 16):
            rows[0][r, pl.ds(v * 16, 16)] = zero16
        return carry

    lax.fori_loop(0, CHUNK, _zrow, 0)
    for k in range(DRAIN_CHUNKS):
        pltpu.sync_copy(
            rows[0],
            accum_sh.at[pl.ds(s * ROWS_PER_TILE + k * CHUNK, CHUNK)])
    plsc.subcore_barrier()

    def idx_start(g, q):
        pltpu.async_copy(
            edata_hbm.at[pl.ds((s * CPT + g) * REC, REC)], ebuf[q], si[q])

    def idx_wait(g, q):
        pltpu.make_async_copy(
            edata_hbm.at[pl.ds((s * CPT + g) * REC, REC)],
            ebuf[q], si[q]).wait()

    def convert(q):
        for j in range(CHUNK // 16):
            sl = pl.ds(j * 16, 16)
            srcc[q][sl] = ebuf[q][pl.ds(j * 16, 16)] + cN
            dstc[q][sl] = ebuf[q][pl.ds(CHUNK + j * 16, 16)]
            r16 = ebuf[q][pl.ds(2 * CHUNK + j * 16, 16)]
            alpc[q][sl] = plsc.load_gather(alpha_v, [r16])

    def gather_start(b, q):
        pltpu.async_copy(support_hbm.at[srcc[q]], rows[b], sg[b])

    def gather_wait(b, q):
        pltpu.make_async_copy(support_hbm.at[srcc[q]], rows[b], sg[b]).wait()

    def scatter_start(b, q):
        pltpu.async_copy(scaled[b], accum_sh.at[dstc[q]], ss[b], add=True)

    def scatter_wait(b, q):
        pltpu.make_async_copy(scaled[b], accum_sh.at[dstc[q]], ss[b]).wait()

    def scale(b, q):
        def _scale(i, c2, _b=b, _q=q):
            for e in range(16):
                idx = i * 16 + e
                spl = plsc.load_gather(
                    alpc[_q], [jnp.broadcast_to(idx, (16,))])
                for v in range(HALF // 16):
                    sl = pl.ds(v * 16, 16)
                    scaled[_b][idx, sl] = rows[_b][idx, sl] * spl
            return c2

        lax.fori_loop(0, CHUNK // 16, _scale, 0)

    def do_iter(g, j, pref, pf, ws_static):
        b, q, q1 = j % 2, j % 4, (j + 1) % 4
        gather_wait(b, q)
        if pref:
            idx_wait(g + 1, q1)
            convert(q1)
            gather_start(1 - b, q1)
        if ws_static:
            scatter_wait(b, (j - 2) % 4)
        else:
            @pl.when(g >= 2)
            def _ws():
                scatter_wait(b, (j - 2) % 4)
        scale(b, q)
        scatter_start(b, q)
        if pf:
            idx_start(g + 2, (j + 2) % 4)

    idx_start(0, 0)
    idx_start(1, 1)
    idx_wait(0, 0)
    convert(0)
    gather_start(0, 0)

    def _outer(gi, cc):
        for j in range(4):
            do_iter(gi * 4 + j, j, pref=True, pf=True, ws_static=j >= 2)
        return cc

    lax.fori_loop(0, (CPT - 2) // 4, _outer, 0)
    do_iter(CPT - 2, 0, pref=True, pf=False, ws_static=True)
    do_iter(CPT - 1, 1, pref=False, pf=False, ws_static=True)
    scatter_wait(0, 0)
    scatter_wait(1, 1)
    plsc.subcore_barrier()

    for k in range(DRAIN_CHUNKS):
        row0 = s * ROWS_PER_TILE + k * CHUNK
        pltpu.sync_copy(accum_sh.at[pl.ds(row0, CHUNK)], rows[0])

        def _brow(r, carry):
            for v in range(HALF // 16):
                sl = pl.ds(v * 16, 16)
                rows[0][r, sl] = rows[0][r, sl] + bias_v[sl]
            return carry

        lax.fori_loop(0, CHUNK, _brow, 0)
        pltpu.sync_copy(rows[0], out_hbm.at[pl.ds(c * NPAD + row0, CHUNK)])


@functools.partial(
    pl.kernel,
    mesh=plsc.VectorSubcoreMesh(core_axis_name="c", subcore_axis_name="s"),
    out_type=jax.ShapeDtypeStruct((2 * NPAD, HALF), jnp.float32),
    compiler_params=pltpu.CompilerParams(needs_layout_passes=False),
    scratch_types=[
        pltpu.VMEM((32,), jnp.float32),
        pltpu.VMEM((HALF,), jnp.float32),
        [pltpu.VMEM((REC,), jnp.int32)] * 4,
        [pltpu.VMEM((CHUNK,), jnp.int32)] * 4,
        [pltpu.VMEM((CHUNK,), jnp.int32)] * 4,
        [pltpu.VMEM((CHUNK,), jnp.float32)] * 4,
        [pltpu.VMEM((CHUNK, HALF), jnp.float32)] * 2,
        [pltpu.VMEM((CHUNK, HALF), jnp.float32)] * 2,
        pltpu.VMEM_SHARED((NPAD, HALF), jnp.float32),
        [pltpu.SemaphoreType.DMA] * 4,
        [pltpu.SemaphoreType.DMA] * 2,
        [pltpu.SemaphoreType.DMA] * 2,
    ],
)
def _sc_scatter(support_hbm, edata_hbm, alpha_hbm, bias_hbm, out_hbm,
                *scratch):
    _sc_body(support_hbm, edata_hbm, alpha_hbm, bias_hbm, out_hbm, *scratch)


def kernel(input, edge_index, rel, weight, alpha_table, bias):
    support_cat = _support_halves(input, weight)
    row, col = edge_index[0], edge_index[1]
    src2 = jnp.concatenate([col, row]).reshape(-1, CHUNK)
    dst2 = jnp.concatenate([row, col]).reshape(-1, CHUNK)
    rel2 = jnp.concatenate([rel, rel]).reshape(-1, CHUNK)
    edata = jnp.stack([src2, dst2, rel2], axis=1).reshape(-1)
    alpha_flat = jnp.pad(alpha_table[:, 0], (0, 32 - alpha_table.shape[0]))
    out2 = _sc_scatter(support_cat, edata, alpha_flat, bias)
    return jnp.concatenate([out2[:N_NODES], out2[NPAD:NPAD + N_NODES]],
                           axis=1)


# speedup vs baseline: 5.4227x; 1.0057x over previous
import functools

import jax
import jax.numpy as jnp
from jax import lax
from jax.experimental import pallas as pl
from jax.experimental.pallas import tpu as pltpu
from jax.experimental.pallas import tpu_sc as plsc

N_NODES = 10000
DIN = 256
DOUT = 256
HALF = DOUT // 2
N_EDGES = 160000

NTILES = 16
E2 = 2 * N_EDGES
PER_TILE = E2 // NTILES
CHUNK = 80
CPT = PER_TILE // CHUNK
REC = 3 * CHUNK
NPAD = 10240
ROWS_PER_TILE = NPAD // NTILES
DRAIN_CHUNKS = ROWS_PER_TILE // CHUNK


def _matmul_body(x_ref, w_ref, o_ref):
    o_ref[...] = jnp.dot(x_ref[...], w_ref[...],
                         preferred_element_type=jnp.float32)


def _support_halves(x, w):
    bn = 1000
    return pl.pallas_call(
        _matmul_body,
        grid=(2, N_NODES // bn),
        in_specs=[
            pl.BlockSpec((bn, DIN), lambda h, i: (i, 0)),
            pl.BlockSpec((DIN, HALF), lambda h, i: (0, h)),
        ],
        out_specs=pl.BlockSpec((bn, HALF),
                               lambda h, i: (h * (N_NODES // bn) + i, 0)),
        out_shape=jax.ShapeDtypeStruct((2 * N_NODES, HALF), jnp.float32),
    )(x, w)


def _sc_body(support_hbm, edata_hbm, alpha_hbm, bias_hbm, out_hbm,
             alpha_v, bias_v, ebuf, srcc, dstc, alpc, rows, accum_sh,
             si, sg, ss):
    c = lax.axis_index("c")
    s = lax.axis_index("s")
    cN = c * N_NODES

    pltpu.sync_copy(alpha_hbm, alpha_v)
    pltpu.sync_copy(bias_hbm.at[pl.ds(c * HALF, HALF)], bias_v)

    lane = lax.iota(jnp.int32, 16)
    a16 = alpha_v[pl.ds(0, 16)]
    alpha_v[pl.ds(0, 16)] = jnp.where(lane == 0,
                                      jnp.zeros((16,), jnp.float32), a16)

    zero16 = jnp.zeros((16,), jnp.float32)

    def _zrow(r, carry):
        for v in range(HALF // 16):
            rows[0][r, pl.ds(v * 16, 16)] = zero16
        return carry

    lax.fori_loop(0, CHUNK, _zrow, 0)
    for k in range(DRAIN_CHUNKS):
        pltpu.sync_copy(
            rows[0],
            accum_sh.at[pl.ds(s * ROWS_PER_TILE + k * CHUNK, CHUNK)])
    plsc.subcore_barrier()

    def idx_start(g, q):
        pltpu.async_copy(
            edata_hbm.at[pl.ds((s * CPT + g) * REC, REC)], ebuf[q], si[q])

    def idx_wait(g, q):
        pltpu.make_async_copy(
            edata_hbm.at[pl.ds((s * CPT + g) * REC, REC)],
            ebuf[q], si[q]).wait()

    def convert(q):
        for j in range(CHUNK // 16):
            sl = pl.ds(j * 16, 16)
            srcc[q][sl] = ebuf[q][pl.ds(j * 16, 16)] + cN
            dstc[q][sl] = ebuf[q][pl.ds(CHUNK + j * 16, 16)]
            r16 = ebuf[q][pl.ds(2 * CHUNK + j * 16, 16)]
            alpc[q][sl] = plsc.load_gather(alpha_v, [r16])

    def gather_start(q):
        pltpu.async_copy(support_hbm.at[srcc[q]], rows[q], sg[q])

    def gather_wait(q):
        pltpu.make_async_copy(support_hbm.at[srcc[q]], rows[q], sg[q]).wait()

    def scatter_start(q):
        pltpu.async_copy(rows[q], accum_sh.at[dstc[q]], ss[q], add=True)

    def scatter_wait(q):
        pltpu.make_async_copy(rows[q], accum_sh.at[dstc[q]], ss[q]).wait()

    def scale(q):
        def _scale(i, c2, _q=q):
            for e in range(16):
                idx = i * 16 + e
                spl = plsc.load_gather(
                    alpc[_q], [jnp.broadcast_to(idx, (16,))])
                for v in range(HALF // 16):
                    sl = pl.ds(v * 16, 16)
                    rows[_q][idx, sl] = rows[_q][idx, sl] * spl
            return c2

        lax.fori_loop(0, CHUNK // 16, _scale, 0)

    def do_iter(g, j, pref, pf, ws_static):
        q, q2, q3 = j % 4, (j + 2) % 4, (j + 3) % 4
        gather_wait(q)
        if ws_static:
            scatter_wait(q2)
        else:
            @pl.when(g >= 2)
            def _ws():
                scatter_wait(q2)
        if pref:
            idx_wait(g + 2, q2)
            convert(q2)
            gather_start(q2)
        scale(q)
        scatter_start(q)
        if pf is True:
            idx_start(g + 3, q3)
        elif pf is not False:
            @pl.when(g + 3 < CPT)
            def _pf():
                idx_start(g + 3, q3)

    idx_start(0, 0)
    idx_start(1, 1)
    idx_start(2, 2)
    idx_wait(0, 0)
    convert(0)
    gather_start(0)
    idx_wait(1, 1)
    convert(1)
    gather_start(1)

    def _outer(gi, cc):
        for j in range(4):
            g = gi * 4 + j
            do_iter(g, j, pref=True, pf=(True if j < 3 else None),
                    ws_static=j >= 2)
        return cc

    lax.fori_loop(0, (CPT - 2) // 4, _outer, 0)
    do_iter(CPT - 2, 0, pref=False, pf=False, ws_static=True)
    do_iter(CPT - 1, 1, pref=False, pf=False, ws_static=True)
    scatter_wait(0)
    scatter_wait(1)
    plsc.subcore_barrier()

    for k in range(DRAIN_CHUNKS):
        row0 = s * ROWS_PER_TILE + k * CHUNK
        pltpu.sync_copy(accum_sh.at[pl.ds(row0, CHUNK)], rows[0])

        def _brow(r, carry):
            for v in range(HALF // 16):
                sl = pl.ds(v * 16, 16)
                rows[0][r, sl] = rows[0][r, sl] + bias_v[sl]
            return carry

        lax.fori_loop(0, CHUNK, _brow, 0)
        pltpu.sync_copy(rows[0], out_hbm.at[pl.ds(c * NPAD + row0, CHUNK)])


@functools.partial(
    pl.kernel,
    mesh=plsc.VectorSubcoreMesh(core_axis_name="c", subcore_axis_name="s"),
    out_type=jax.ShapeDtypeStruct((2 * NPAD, HALF), jnp.float32),
    compiler_params=pltpu.CompilerParams(needs_layout_passes=False),
    scratch_types=[
        pltpu.VMEM((32,), jnp.float32),
        pltpu.VMEM((HALF,), jnp.float32),
        [pltpu.VMEM((REC,), jnp.int32)] * 4,
        [pltpu.VMEM((CHUNK,), jnp.int32)] * 4,
        [pltpu.VMEM((CHUNK,), jnp.int32)] * 4,
        [pltpu.VMEM((CHUNK,), jnp.float32)] * 4,
        [pltpu.VMEM((CHUNK, HALF), jnp.float32)] * 4,
        pltpu.VMEM_SHARED((NPAD, HALF), jnp.float32),
        [pltpu.SemaphoreType.DMA] * 4,
        [pltpu.SemaphoreType.DMA] * 4,
        [pltpu.SemaphoreType.DMA] * 4,
    ],
)
def _sc_scatter(support_hbm, edata_hbm, alpha_hbm, bias_hbm, out_hbm,
                *scratch):
    _sc_body(support_hbm, edata_hbm, alpha_hbm, bias_hbm, out_hbm, *scratch)


def kernel(input, edge_index, rel, weight, alpha_table, bias):
    support_cat = _support_halves(input, weight)
    row, col = edge_index[0], edge_index[1]
    src2 = jnp.concatenate([col, row]).reshape(-1, CHUNK)
    dst2 = jnp.concatenate([row, col]).reshape(-1, CHUNK)
    rel2 = jnp.concatenate([rel, rel]).reshape(-1, CHUNK)
    edata = jnp.stack([src2, dst2, rel2], axis=1).reshape(-1)
    alpha_flat = jnp.pad(alpha_table[:, 0], (0, 32 - alpha_table.shape[0]))
    out2 = _sc_scatter(support_cat, edata, alpha_flat, bias)
    return jnp.concatenate([out2[:N_NODES], out2[NPAD:NPAD + N_NODES]],
                           axis=1)
